# baseline (speedup 1.0000x reference)
import functools

import jax
import jax.numpy as jnp
from jax import lax
from jax.experimental import pallas as pl
from jax.experimental.pallas import tpu as pltpu
from jax.experimental.pallas import tpu_sc as plsc

HD = 192
ED = 64
NBATCH = 1024
PP = 2
TB = PP * NBATCH
CH = 128


def _sc_gather_pairs(tables, idxs, T):
    info = plsc.get_sparse_core_info()
    NC, NS = info.num_cores, info.num_subcores
    NW = NC * NS
    per_w = T // NW
    n_ch = per_w // CH
    mesh = plsc.VectorSubcoreMesh(core_axis_name="c", subcore_axis_name="s")

    @functools.partial(
        pl.kernel,
        mesh=mesh,
        compiler_params=pltpu.CompilerParams(use_tc_tiling_on_sc=False),
        out_type=[jax.ShapeDtypeStruct((T, 2 * ED), jnp.float32) for _ in range(3)],
        scratch_types=(
            [pltpu.VMEM((CH,), jnp.int32) for _ in range(6)]
            + [pltpu.VMEM((CH, ED), jnp.float32) for _ in range(6)]
            + [pltpu.SemaphoreType.DMA, pltpu.SemaphoreType.DMA,
               pltpu.SemaphoreType.DMA]
        ),
    )
    def k(t0, t1, t2, t3, t4, t5, i0, i1, i2, i3, i4, i5,
          oa, ob, oc, v0, v1, v2, v3, v4, v5, r0, r1, r2, r3, r4, r5,
          sem_i, sem_g, sem_w):
        tabs = (t0, t1, t2, t3, t4, t5)
        idxv = (v0, v1, v2, v3, v4, v5)
        rows = (r0, r1, r2, r3, r4, r5)
        outs = (oa, ob, oc)
        wid = lax.axis_index("s") * NC + lax.axis_index("c")
        base = wid * per_w

        def body(c, carry):
            off = base + c * CH
            loads = [
                pltpu.async_copy(idx.at[pl.ds(off, CH)], v, sem_i)
                for idx, v in zip((i0, i1, i2, i3, i4, i5), idxv)
            ]
            for cp in loads:
                cp.wait()
            gathers = [
                pltpu.async_copy(tabs[f].at[idxv[f]], rows[f], sem_g)
                for f in range(6)
            ]
            for cp in gathers:
                cp.wait()
            writes = [
                pltpu.async_copy(
                    rows[f],
                    outs[f // 2].at[pl.ds(off, CH), pl.ds((f % 2) * ED, ED)],
                    sem_w)
                for f in range(6)
            ]
            for cp in writes:
                cp.wait()
            return carry

        lax.fori_loop(0, n_ch, body, 0)

    return k(*tables, *idxs)


def _tc_body(el_r, td_r, pa, pb, pc, wa, wb, wc, wea, web,
             wcel, wctd, weel, wetd, bc, be, eo, xo):
    a = pa[...]
    b = pb[...]
    c = pc[...]
    f32 = jnp.float32
    x = jnp.dot(a, wa[...], preferred_element_type=f32)
    x += jnp.dot(b, wb[...], preferred_element_type=f32)
    x += jnp.dot(c, wc[...], preferred_element_type=f32)
    e = jnp.dot(a, wea[...], preferred_element_type=f32)
    e += jnp.dot(b, web[...], preferred_element_type=f32)
    el = el_r[0, 0, :][:, None]
    td = td_r[0, 0, :][:, None]
    x += el * wcel[...]
    x += td * wctd[...]
    x += bc[...]
    e += el * weel[...]
    e += td * wetd[...]
    e += be[...]
    for p in range(PP):
        xo[p] = x[p * NBATCH:(p + 1) * NBATCH, :].T
        eo[p] = e[p * NBATCH:(p + 1) * NBATCH, :].T


def _tc_project(el3, td3, pa, pb, pc, wblocks, T, S):
    NB = T // TB
    pair_spec = pl.BlockSpec((TB, 2 * ED), lambda i: (i, 0))
    tok_spec = pl.BlockSpec((1, 1, TB), lambda i: (i, 0, 0))
    full = lambda s: pl.BlockSpec(s, lambda i: (0, 0))
    in_specs = (
        [tok_spec, tok_spec, pair_spec, pair_spec, pair_spec]
        + [full(w.shape) for w in wblocks]
    )
    out_specs = [pl.BlockSpec((PP, HD, NBATCH), lambda i: (i, 0, 0))] * 2
    out_shape = [jax.ShapeDtypeStruct((S, HD, NBATCH), jnp.float32)] * 2
    return pl.pallas_call(
        _tc_body,
        grid=(NB,),
        in_specs=in_specs,
        out_specs=out_specs,
        out_shape=out_shape,
    )(el3, td3, pa, pb, pc, *wblocks)


def kernel(interaction, user_idx, item_idx, assessmentItemID, testId, KnowledgeTag,
           elapsed, time_diff, user_emb, item_emb, emb_interaction, emb_assess,
           emb_test, emb_tag, W_comb, b_comb, W_enc, b_enc):
    B, S = interaction.shape
    T = B * S
    NB = T // TB

    i32 = jnp.int32
    idx_n = interaction.T.reshape(-1).astype(i32)
    idx_a = assessmentItemID.T.reshape(-1).astype(i32)
    idx_t = testId.T.reshape(-1).astype(i32)
    idx_g = KnowledgeTag.T.reshape(-1).astype(i32)
    idx_u = user_idx.T.reshape(-1).astype(i32)
    idx_i = item_idx.T.reshape(-1).astype(i32)

    emb_inter8 = jnp.concatenate(
        [emb_interaction, jnp.zeros((5, ED), jnp.float32)], axis=0)
    pa, pb, pc = _sc_gather_pairs(
        (emb_inter8, emb_assess, emb_test, emb_tag, user_emb, item_emb),
        (idx_n, idx_a, idx_t, idx_g, idx_u, idx_i), T)

    el3 = elapsed.T.reshape(NB, 1, TB)
    td3 = time_diff.T.reshape(NB, 1, TB)

    wblocks = (
        W_comb[0:128],
        W_comb[128:256],
        W_comb[258:386],
        jnp.concatenate([jnp.zeros((ED, HD), jnp.float32),
                         W_enc[0:64]], axis=0),
        W_enc[64:192],
        W_comb[256:257],
        W_comb[257:258],
        W_enc[192:193],
        W_enc[193:194],
        b_comb.reshape(1, HD),
        b_enc.reshape(1, HD),
    )
    enc_x, x = _tc_project(el3, td3, pa, pb, pc, wblocks, T, S)
    return (jnp.transpose(enc_x, (2, 0, 1)), jnp.transpose(x, (2, 0, 1)))
